# baseline (speedup 1.0000x reference)
import jax
import jax.numpy as jnp
from jax import lax
from jax.experimental import pallas as pl
from jax.experimental.pallas import tpu as pltpu

_CHUNK = 512
_NBUF = 8


def _router_body(x_hbm, w_ref, probs_ref, logits_ref, xbuf, in_sem):
    n_tokens = x_hbm.shape[0]
    n_chunks = n_tokens // _CHUNK
    w = w_ref[...]

    def in_copy(chunk, buf):
        return pltpu.make_async_copy(
            x_hbm.at[pl.ds(chunk * _CHUNK, _CHUNK), :],
            xbuf.at[buf],
            in_sem.at[buf],
        )

    for b in range(_NBUF):
        in_copy(b, b).start(priority=1)

    def step(j, carry):
        i0 = _NBUF * j
        for k in range(_NBUF):
            i = i0 + k
            in_copy(i, k).wait()

            x = xbuf[k]
            logits = jnp.dot(x, w, preferred_element_type=jnp.float32)
            m = jnp.max(logits, axis=-1, keepdims=True)
            e = jnp.exp(logits - m)
            probs = e / jnp.sum(e, axis=-1, keepdims=True)
            probs_ref[pl.ds(i * _CHUNK, _CHUNK), :] = probs
            logits_ref[pl.ds(i * _CHUNK, _CHUNK), :] = logits

            @pl.when(i + _NBUF < n_chunks)
            def _():
                in_copy(i + _NBUF, k).start(priority=1)

        return carry

    lax.fori_loop(0, n_chunks // _NBUF, step, 0)


def kernel(inputs, num_experts, w):
    n_tokens, d = inputs.shape
    n_exp = w.shape[1]
    probs, logits = pl.pallas_call(
        _router_body,
        in_specs=[
            pl.BlockSpec(memory_space=pl.ANY),
            pl.BlockSpec(memory_space=pltpu.VMEM),
        ],
        out_specs=[
            pl.BlockSpec(memory_space=pltpu.VMEM),
            pl.BlockSpec(memory_space=pltpu.VMEM),
        ],
        out_shape=[
            jax.ShapeDtypeStruct((n_tokens, n_exp), jnp.float32),
            jax.ShapeDtypeStruct((n_tokens, n_exp), jnp.float32),
        ],
        scratch_shapes=[
            pltpu.VMEM((_NBUF, _CHUNK, d), jnp.float32),
            pltpu.SemaphoreType.DMA((_NBUF,)),
        ],
    )(inputs, w)
    return (probs, logits, 0)


# speedup vs baseline: 1.2684x; 1.2684x over previous
import jax
import jax.numpy as jnp
from jax import lax
from jax.experimental import pallas as pl
from jax.experimental.pallas import tpu as pltpu

_CHUNK = 1024
_NBUF = 8


def _router_body(x_hbm, w_ref, probs_hbm, logits_hbm,
                 xbuf, pbuf, lbuf, in_sem, p_sem, l_sem):
    n_tokens = x_hbm.shape[0]
    n_chunks = n_tokens // _CHUNK
    w = w_ref[...]

    def in_copy(chunk, buf):
        return pltpu.make_async_copy(
            x_hbm.at[pl.ds(chunk * _CHUNK, _CHUNK), :],
            xbuf.at[buf],
            in_sem.at[buf],
        )

    for b in range(_NBUF):
        in_copy(b, b).start(priority=1)

    def step(j, carry):
        i0 = _NBUF * j
        for k in range(_NBUF):
            i = i0 + k
            in_copy(i, k).wait()

            @pl.when(j >= 1)
            def _():
                pltpu.make_async_copy(
                    pbuf.at[k], probs_hbm.at[pl.ds(0, _CHUNK), :], p_sem.at[k]
                ).wait()
                pltpu.make_async_copy(
                    lbuf.at[k], logits_hbm.at[pl.ds(0, _CHUNK), :], l_sem.at[k]
                ).wait()

            x = xbuf[k]
            logits = jnp.dot(x, w, preferred_element_type=jnp.float32)
            m = jnp.max(logits, axis=-1, keepdims=True)
            e = jnp.exp(logits - m)
            probs = e / jnp.sum(e, axis=-1, keepdims=True)
            pbuf[k] = probs
            lbuf[k] = logits

            pltpu.make_async_copy(
                pbuf.at[k], probs_hbm.at[pl.ds(i * _CHUNK, _CHUNK), :],
                p_sem.at[k]
            ).start()
            pltpu.make_async_copy(
                lbuf.at[k], logits_hbm.at[pl.ds(i * _CHUNK, _CHUNK), :],
                l_sem.at[k]
            ).start()

            @pl.when(i + _NBUF < n_chunks)
            def _():
                in_copy(i + _NBUF, k).start(priority=1)

        return carry

    lax.fori_loop(0, n_chunks // _NBUF, step, 0)

    for k in range(_NBUF):
        pltpu.make_async_copy(
            pbuf.at[k], probs_hbm.at[pl.ds(0, _CHUNK), :], p_sem.at[k]
        ).wait()
        pltpu.make_async_copy(
            lbuf.at[k], logits_hbm.at[pl.ds(0, _CHUNK), :], l_sem.at[k]
        ).wait()


def kernel(inputs, num_experts, w):
    n_tokens, d = inputs.shape
    n_exp = w.shape[1]
    probs, logits = pl.pallas_call(
        _router_body,
        in_specs=[
            pl.BlockSpec(memory_space=pl.ANY),
            pl.BlockSpec(memory_space=pltpu.VMEM),
        ],
        out_specs=[
            pl.BlockSpec(memory_space=pl.ANY),
            pl.BlockSpec(memory_space=pl.ANY),
        ],
        out_shape=[
            jax.ShapeDtypeStruct((n_tokens, n_exp), jnp.float32),
            jax.ShapeDtypeStruct((n_tokens, n_exp), jnp.float32),
        ],
        scratch_shapes=[
            pltpu.VMEM((_NBUF, _CHUNK, d), jnp.float32),
            pltpu.VMEM((_NBUF, _CHUNK, n_exp), jnp.float32),
            pltpu.VMEM((_NBUF, _CHUNK, n_exp), jnp.float32),
            pltpu.SemaphoreType.DMA((_NBUF,)),
            pltpu.SemaphoreType.DMA((_NBUF,)),
            pltpu.SemaphoreType.DMA((_NBUF,)),
        ],
    )(inputs, w)
    return (probs, logits, 0)
